# baseline (speedup 1.0000x reference)
import functools

import jax
import jax.numpy as jnp
from jax import lax
from jax.experimental import pallas as pl
from jax.experimental.pallas import tpu as pltpu
from jax.experimental.pallas import tpu_sc as plsc

_BATCH = 16384
_DIM = 64
_NUM_CORES = 2
_NUM_SUBCORES = 16
_NW = _NUM_CORES * _NUM_SUBCORES
_BPW = _BATCH // _NW
_LANES = 16
_CHUNK = 256
_NCHUNK = _BPW // _CHUNK

_mesh = plsc.VectorSubcoreMesh(core_axis_name="c", subcore_axis_name="s")


@functools.partial(
    pl.kernel,
    mesh=_mesh,
    out_type=jax.ShapeDtypeStruct((_BATCH, _DIM), jnp.float32),
    scratch_types=[
        pltpu.VMEM((_BPW,), jnp.int32),
        pltpu.VMEM((_BPW,), jnp.int32),
        pltpu.VMEM((_CHUNK, 2 * _DIM), jnp.float32),
        pltpu.VMEM((_CHUNK, _DIM), jnp.float32),
        pltpu.SemaphoreType.DMA,
    ],
)
def _embedding_gather(idx_hbm, pairs_hbm, out_hbm,
                      idx_v, pidx_v, rows_v, out_v, sem):
    wid = lax.axis_index("s") * _NUM_CORES + lax.axis_index("c")
    base = wid * _BPW
    pltpu.sync_copy(idx_hbm.at[pl.ds(base, _BPW)], idx_v)

    def halve(j, carry):
        sl = pl.ds(j * _LANES, _LANES)
        pidx_v[sl] = idx_v[sl] >> 1
        return carry

    lax.fori_loop(0, _BPW // _LANES, halve, 0, unroll=4)

    for k in range(_NCHUNK):
        pltpu.async_copy(
            pairs_hbm.at[pidx_v.at[pl.ds(k * _CHUNK, _CHUNK)]], rows_v, sem
        ).wait()

        def select(g, carry, k=k):
            hv = (idx_v[pl.ds(k * _CHUNK + g * _LANES, _LANES)] & 1) * _DIM
            for r in range(_LANES):
                col = hv[r]
                i = g * _LANES + r
                for j in range(_DIM // _LANES):
                    out_v[i, pl.ds(j * _LANES, _LANES)] = (
                        rows_v[i, pl.ds(col + j * _LANES, _LANES)]
                    )
            return carry

        lax.fori_loop(0, _CHUNK // _LANES, select, 0)
        pltpu.sync_copy(out_v, out_hbm.at[pl.ds(base + k * _CHUNK, _CHUNK)])


def kernel(c, table):
    pairs = table.reshape(500000, 2 * _DIM)
    return _embedding_gather(c.astype(jnp.int32), pairs)
